# baseline (speedup 1.0000x reference)
import jax
import jax.numpy as jnp
from jax.experimental import pallas as pl
from jax.experimental.pallas import tpu as pltpu

_ROWS = 393600
_D = 64
_NCHUNK = 40
_CH = _ROWS // _NCHUNK
_K = 4


def _copy_body(x_ref, o_ref, buf, in_sems, out_sems):
    xf = x_ref.reshape(_ROWS, _D)
    of = o_ref.reshape(_ROWS, _D)

    def src(i):
        return xf.at[pl.ds(i * _CH, _CH)]

    def dst(i):
        return of.at[pl.ds(i * _CH, _CH)]

    for k in range(_K):
        pltpu.make_async_copy(src(k), buf.at[k], in_sems.at[k]).start(
            priority=0
        )

    for g in range(_NCHUNK // _K):
        base = g * _K
        for k in range(_K):
            i = base + k
            pltpu.make_async_copy(src(i), buf.at[k], in_sems.at[k]).wait()
            pltpu.make_async_copy(buf.at[k], dst(i), out_sems.at[k]).start(
                priority=1
            )
        for k in range(_K):
            i = base + k
            pltpu.make_async_copy(buf.at[k], dst(i), out_sems.at[k]).wait()
            if i + _K < _NCHUNK:
                pltpu.make_async_copy(
                    src(i + _K), buf.at[k], in_sems.at[k]
                ).start(priority=0)


def kernel(x):
    return pl.pallas_call(
        _copy_body,
        in_specs=[pl.BlockSpec(memory_space=pl.ANY)],
        out_specs=pl.BlockSpec(memory_space=pl.ANY),
        out_shape=jax.ShapeDtypeStruct(x.shape, x.dtype),
        scratch_shapes=[
            pltpu.VMEM((_K, _CH, _D), jnp.float32),
            pltpu.SemaphoreType.DMA((_K,)),
            pltpu.SemaphoreType.DMA((_K,)),
        ],
    )(x)


# speedup vs baseline: 1.0005x; 1.0005x over previous
import jax
import jax.numpy as jnp
from jax.experimental import pallas as pl
from jax.experimental.pallas import tpu as pltpu

_ROWS = 393600
_D = 64
_NCORE = 2
_NCHUNK = 40
_PER_CORE = _NCHUNK // _NCORE
_CH = _ROWS // _NCHUNK
_K = 4


def _copy_body(x_ref, o_ref, buf, in_sems, out_sems):
    xf = x_ref.reshape(_ROWS, _D)
    of = o_ref.reshape(_ROWS, _D)
    first = pl.program_id(0) * _PER_CORE

    def src(i):
        return xf.at[pl.ds((first + i) * _CH, _CH)]

    def dst(i):
        return of.at[pl.ds((first + i) * _CH, _CH)]

    for k in range(_K):
        pltpu.make_async_copy(src(k), buf.at[k], in_sems.at[k]).start(
            priority=0
        )

    for g in range(_PER_CORE // _K):
        base = g * _K
        for k in range(_K):
            i = base + k
            pltpu.make_async_copy(src(i), buf.at[k], in_sems.at[k]).wait()
            pltpu.make_async_copy(buf.at[k], dst(i), out_sems.at[k]).start(
                priority=1
            )
        for k in range(_K):
            i = base + k
            pltpu.make_async_copy(buf.at[k], dst(i), out_sems.at[k]).wait()
            if i + _K < _PER_CORE:
                pltpu.make_async_copy(
                    src(i + _K), buf.at[k], in_sems.at[k]
                ).start(priority=0)


def kernel(x):
    return pl.pallas_call(
        _copy_body,
        grid=(_NCORE,),
        in_specs=[pl.BlockSpec(memory_space=pl.ANY)],
        out_specs=pl.BlockSpec(memory_space=pl.ANY),
        out_shape=jax.ShapeDtypeStruct(x.shape, x.dtype),
        scratch_shapes=[
            pltpu.VMEM((_K, _CH, _D), jnp.float32),
            pltpu.SemaphoreType.DMA((_K,)),
            pltpu.SemaphoreType.DMA((_K,)),
        ],
        compiler_params=pltpu.CompilerParams(
            dimension_semantics=("parallel",),
        ),
    )(x)
